# baseline (speedup 1.0000x reference)
import functools

import jax
import jax.numpy as jnp
from jax import lax
from jax.experimental import pallas as pl
from jax.experimental.pallas import tpu as pltpu
from jax.experimental.pallas import tpu_sc as plsc

N = 10000
D = 128
E = 320000
NC, NS = 2, 16
NW = NC * NS
EPT = E // NW
CH = 80
NCH = EPT // CH
PAD = 10240
STRIPE = PAD // NS
BLK = 1000
GRID = N // BLK


_MESH = plsc.VectorSubcoreMesh(core_axis_name="c", subcore_axis_name="s",
                               num_cores=NC, num_subcores=NS)

NBUF = 5


def _make_ring_agg(width, gather):
    vec = (D,) if width == D else ()
    scratch = [pltpu.VMEM((NCH, CH), jnp.int32)]
    if gather:
        scratch.insert(0, pltpu.VMEM((EPT,), jnp.int32))
        scratch.append(pltpu.VMEM((PAD,), jnp.float32))
    nrows = NBUF if gather else 1
    scratch += [pltpu.VMEM((CH,) + vec, jnp.float32) for _ in range(nrows)]
    scratch.append(pltpu.VMEM_SHARED((PAD,) + vec, jnp.float32))
    nsem = NBUF if gather else 1
    scratch += [pltpu.SemaphoreType.DMA for _ in range(nsem)]

    def body(*refs):
        if gather:
            (src_hbm, dst_hbm, table_hbm, zeros_hbm, out_hbm, sidx, didx,
             table_vm, *rest) = refs
            rows = rest[:NBUF]
            acc = rest[NBUF]
            ssem = rest[NBUF + 1:]
        else:
            (dst_hbm, zeros_hbm, out_hbm, didx, rows1, acc, ssem1) = refs
        c = lax.axis_index("c")
        s = lax.axis_index("s")
        wid = c * NS + s
        pltpu.sync_copy(zeros_hbm.at[pl.ds(s * STRIPE, STRIPE)],
                        acc.at[pl.ds(s * STRIPE, STRIPE)])
        pltpu.sync_copy(dst_hbm.at[wid], didx)

        if gather:
            pltpu.sync_copy(src_hbm.at[pl.ds(wid * EPT, EPT)], sidx)
            pltpu.sync_copy(table_hbm, table_vm)
            plsc.subcore_barrier()

            def ring(r, carry):
                for b in range(NBUF):
                    j = r * NBUF + b

                    @pl.when(r > 0)
                    def _():
                        pltpu.make_async_copy(rows[b], acc.at[didx.at[0]],
                                              ssem[b]).wait()
                    for k in range(CH // 16):
                        iv = sidx[pl.ds(j * CH + k * 16, 16)]
                        vals = plsc.load_gather(table_vm, [iv])
                        rows[b][pl.ds(k * 16, 16)] = vals
                    pltpu.async_copy(rows[b], acc.at[didx.at[j]],
                                     ssem[b], add=True)
                return carry

            lax.fori_loop(0, NCH // NBUF, ring, 0)
            for b in range(NBUF):
                pltpu.make_async_copy(rows[b], acc.at[didx.at[0]],
                                      ssem[b]).wait()
        else:
            for i in range(CH // 16):
                rows1[pl.ds(i * 16, 16)] = jnp.ones((16,), jnp.float32)
            plsc.subcore_barrier()

            def fire(j, carry):
                pltpu.async_copy(rows1, acc.at[didx.at[j]], ssem1,
                                 add=True)

                @pl.when(j >= NBUF)
                def _():
                    pltpu.make_async_copy(rows1, acc.at[didx.at[0]],
                                          ssem1).wait()
                return carry

            lax.fori_loop(0, NCH, fire, 0)
            for _ in range(NBUF):
                pltpu.make_async_copy(rows1, acc.at[didx.at[0]],
                                      ssem1).wait()
        plsc.subcore_barrier()
        pltpu.sync_copy(acc.at[pl.ds(s * STRIPE, STRIPE)],
                        out_hbm.at[c, pl.ds(s * STRIPE, STRIPE)])

    return pl.kernel(
        body,
        out_type=jax.ShapeDtypeStruct((NC, PAD) + vec, jnp.float32),
        mesh=_MESH,
        scratch_types=scratch,
        compiler_params=pltpu.CompilerParams(needs_layout_passes=False),
    )


_deg_count = _make_ring_agg(1, gather=False)
_agg_scalar = _make_ring_agg(1, gather=True)

CH2 = 80
NCH2 = EPT // CH2
NB2 = 3
NTAIL = NCH2 - (NCH2 // (2 * NB2)) * 2 * NB2


def _rows_agg_body(idx2_hbm, table_hbm, zeros_hbm, out_hbm, *rest):
    idxA = rest[:NB2]
    idxB = rest[NB2:2 * NB2]
    rows = rest[2 * NB2:3 * NB2]
    acc = rest[3 * NB2]
    r0 = 3 * NB2 + 1
    isemA = rest[r0:r0 + NB2]
    isemB = rest[r0 + NB2:r0 + 2 * NB2]
    gsem = rest[r0 + 2 * NB2:r0 + 3 * NB2]
    ssem = rest[r0 + 3 * NB2:]
    c = lax.axis_index("c")
    s = lax.axis_index("s")
    wid = c * NS + s
    for b in range(NB2):
        pltpu.async_copy(idx2_hbm.at[wid, b], idxA[b], isemA[b])
        pltpu.async_copy(idx2_hbm.at[wid, NB2 + b], idxB[b], isemB[b])
    pltpu.sync_copy(zeros_hbm.at[pl.ds(s * STRIPE, STRIPE)],
                    acc.at[pl.ds(s * STRIPE, STRIPE)])
    for b in range(NB2):
        pltpu.make_async_copy(idx2_hbm.at[wid, 0], idxA[b],
                              isemA[b]).wait()
        pltpu.async_copy(table_hbm.at[idxA[b].at[0]], rows[b], gsem[b])
    plsc.subcore_barrier()

    def ring(i, carry):
        base = i * 2 * NB2
        for b in range(NB2):
            pltpu.make_async_copy(table_hbm.at[idxA[b].at[0]], rows[b],
                                  gsem[b]).wait()
            pltpu.async_copy(rows[b], acc.at[idxA[b].at[1]], ssem[b],
                             add=True)
        for b in range(NB2):
            pltpu.make_async_copy(idx2_hbm.at[wid, 0], idxB[b],
                                  isemB[b]).wait()
            pltpu.make_async_copy(rows[b], acc.at[idxA[b].at[1]],
                                  ssem[b]).wait()
            pltpu.async_copy(table_hbm.at[idxB[b].at[0]], rows[b],
                             gsem[b])
            jf = base + 2 * NB2 + b

            @pl.when(jf < NCH2)
            def _():
                pltpu.async_copy(idx2_hbm.at[wid, jf], idxA[b], isemA[b])
        for b in range(NB2):
            pltpu.make_async_copy(table_hbm.at[idxB[b].at[0]], rows[b],
                                  gsem[b]).wait()
            pltpu.async_copy(rows[b], acc.at[idxB[b].at[1]], ssem[b],
                             add=True)
        for b in range(NB2):
            jn = base + 2 * NB2 + b

            @pl.when(jn < NCH2)
            def _():
                pltpu.make_async_copy(idx2_hbm.at[wid, 0], idxA[b],
                                      isemA[b]).wait()
                pltpu.make_async_copy(rows[b], acc.at[idxB[b].at[1]],
                                      ssem[b]).wait()
                pltpu.async_copy(table_hbm.at[idxA[b].at[0]], rows[b],
                                 gsem[b])
                jf = base + 3 * NB2 + b

                @pl.when(jf < NCH2)
                def _():
                    pltpu.async_copy(idx2_hbm.at[wid, jf], idxB[b],
                                     isemB[b])
        return carry

    lax.fori_loop(0, NCH2 // (2 * NB2), ring, 0)
    for b in range(NB2):
        pltpu.make_async_copy(table_hbm.at[idxA[b].at[0]], rows[b],
                              gsem[b]).wait()
        pltpu.async_copy(rows[b], acc.at[idxA[b].at[1]], ssem[b],
                         add=True)
    for b in range(NTAIL - NB2):
        pltpu.make_async_copy(idx2_hbm.at[wid, 0], idxB[b],
                              isemB[b]).wait()
        pltpu.make_async_copy(rows[b], acc.at[idxA[b].at[1]],
                              ssem[b]).wait()
        pltpu.async_copy(table_hbm.at[idxB[b].at[0]], rows[b], gsem[b])
    for b in range(NTAIL - NB2):
        pltpu.make_async_copy(table_hbm.at[idxB[b].at[0]], rows[b],
                              gsem[b]).wait()
        pltpu.async_copy(rows[b], acc.at[idxB[b].at[1]], ssem[b],
                         add=True)
    for b in range(NTAIL - NB2):
        pltpu.make_async_copy(rows[b], acc.at[idxB[b].at[1]],
                              ssem[b]).wait()
    for b in range(NTAIL - NB2, NB2):
        pltpu.make_async_copy(rows[b], acc.at[idxA[b].at[1]],
                              ssem[b]).wait()
    plsc.subcore_barrier()
    pltpu.sync_copy(acc.at[pl.ds(s * STRIPE, STRIPE)],
                    out_hbm.at[c, pl.ds(s * STRIPE, STRIPE)])


_agg_rows = pl.kernel(
    _rows_agg_body,
    out_type=jax.ShapeDtypeStruct((NC, PAD, D), jnp.float32),
    mesh=_MESH,
    scratch_types=(
        [pltpu.VMEM((2, CH2), jnp.int32) for _ in range(2 * NB2)]
        + [pltpu.VMEM((CH2, D), jnp.float32) for _ in range(NB2)]
        + [pltpu.VMEM_SHARED((PAD, D), jnp.float32)]
        + [pltpu.SemaphoreType.DMA for _ in range(4 * NB2)]
    ),
)


def _tc_mm(x_ref, w1_ref, xw_ref):
    xw_ref[...] = jnp.dot(x_ref[...], w1_ref[...],
                          preferred_element_type=jnp.float32)


def _tc_pre(xw_ref, cnt_ref, xws_ref):
    cnt = cnt_ref[0, 0, 0, :] + cnt_ref[1, 0, 0, :]
    dinv = lax.rsqrt(cnt + 1.0)
    xws_ref[...] = xw_ref[...] * dinv[:, None]


def _tc_mid(p_ref, xw_ref, cnt_ref, b1_ref, w2_ref, wlin_ref, z_ref, zs_ref):
    agg = p_ref[0] + p_ref[1]
    cnt = cnt_ref[0, 0, 0, :] + cnt_ref[1, 0, 0, :]
    deg = cnt + 1.0
    dinv = lax.rsqrt(deg)
    h1 = agg * dinv[:, None] + xw_ref[...] * (1.0 / deg)[:, None] + b1_ref[...]
    h1 = jnp.maximum(h1, 0.0)
    w2l = jnp.dot(w2_ref[...], wlin_ref[...], preferred_element_type=jnp.float32)
    z = jnp.dot(h1, w2l, preferred_element_type=jnp.float32)
    z_ref[...] = z
    zs_ref[...] = z * dinv[:, None]


def _tc_post(az_ref, z_ref, cnt_ref, b2_ref, wlin_ref, blin_ref, out_ref):
    az = az_ref[0, 0, 0, :] + az_ref[1, 0, 0, :]
    cnt = cnt_ref[0, 0, 0, :] + cnt_ref[1, 0, 0, :]
    deg = cnt + 1.0
    dinv = lax.rsqrt(deg)
    c0 = jnp.sum(b2_ref[...] * wlin_ref[..., 0]) + blin_ref[0]
    pre = az * dinv + z_ref[..., 0] * (1.0 / deg) + c0
    out_ref[...] = jax.nn.sigmoid(pre)[:, None]


def kernel(x, edge_index, W1, b1, W2, b2, Wlin, blin):
    ei = edge_index.astype(jnp.int32)
    src_r = ei[0].reshape(NW, NCH, CH)
    dst_r = ei[1].reshape(NW, NCH, CH)
    idx2 = jnp.stack([ei[0].reshape(NW, NCH2, CH2),
                      ei[1].reshape(NW, NCH2, CH2)], axis=2)
    zeros1 = jnp.zeros((PAD,), jnp.float32)
    zerosD = jnp.zeros((PAD, D), jnp.float32)

    xw = pl.pallas_call(
        _tc_mm,
        grid=(GRID,),
        in_specs=[
            pl.BlockSpec((BLK, D), lambda i: (i, 0)),
            pl.BlockSpec((D, D), lambda i: (0, 0)),
        ],
        out_specs=pl.BlockSpec((BLK, D), lambda i: (i, 0)),
        out_shape=jax.ShapeDtypeStruct((N, D), jnp.float32),
    )(x, W1)

    cnt = _deg_count(dst_r, zeros1)
    cnt_r = cnt[:, :N].reshape(NC, GRID, 1, BLK)
    cnt_spec = pl.BlockSpec((NC, 1, 1, BLK), lambda i: (0, i, 0, 0))

    xws = pl.pallas_call(
        _tc_pre,
        grid=(GRID,),
        in_specs=[
            pl.BlockSpec((BLK, D), lambda i: (i, 0)),
            cnt_spec,
        ],
        out_specs=pl.BlockSpec((BLK, D), lambda i: (i, 0)),
        out_shape=jax.ShapeDtypeStruct((N, D), jnp.float32),
    )(xw, cnt_r)

    p = _agg_rows(idx2, xws, zerosD)

    z, zs = pl.pallas_call(
        _tc_mid,
        grid=(GRID,),
        in_specs=[
            pl.BlockSpec((NC, BLK, D), lambda i: (0, i, 0)),
            pl.BlockSpec((BLK, D), lambda i: (i, 0)),
            cnt_spec,
            pl.BlockSpec((D,), lambda i: (0,)),
            pl.BlockSpec((D, D), lambda i: (0, 0)),
            pl.BlockSpec((D, 1), lambda i: (0, 0)),
        ],
        out_specs=[
            pl.BlockSpec((BLK, 1), lambda i: (i, 0)),
            pl.BlockSpec((BLK, 1), lambda i: (i, 0)),
        ],
        out_shape=[
            jax.ShapeDtypeStruct((N, 1), jnp.float32),
            jax.ShapeDtypeStruct((N, 1), jnp.float32),
        ],
    )(p, xw, cnt_r, b1, W2, Wlin)

    zs_pad = jnp.concatenate([zs.reshape(N),
                              jnp.zeros((PAD - N,), jnp.float32)])
    az = _agg_scalar(ei[0], dst_r, zs_pad, zeros1)
    az_r = az[:, :N].reshape(NC, GRID, 1, BLK)

    out = pl.pallas_call(
        _tc_post,
        grid=(GRID,),
        in_specs=[
            cnt_spec,
            pl.BlockSpec((BLK, 1), lambda i: (i, 0)),
            cnt_spec,
            pl.BlockSpec((D,), lambda i: (0,)),
            pl.BlockSpec((D, 1), lambda i: (0, 0)),
            pl.BlockSpec((1,), lambda i: (0,)),
        ],
        out_specs=pl.BlockSpec((BLK, 1), lambda i: (i, 0)),
        out_shape=jax.ShapeDtypeStruct((N, 1), jnp.float32),
    )(az_r, z, cnt_r, b2, Wlin, blin)

    return out


# speedup vs baseline: 1.0269x; 1.0269x over previous
import functools

import jax
import jax.numpy as jnp
from jax import lax
from jax.experimental import pallas as pl
from jax.experimental.pallas import tpu as pltpu
from jax.experimental.pallas import tpu_sc as plsc

N = 10000
D = 128
E = 320000
NC, NS = 2, 16
NW = NC * NS
EPT = E // NW
CH = 80
NCH = EPT // CH
PAD = 10240
STRIPE = PAD // NS
BLK = 1000
GRID = N // BLK


_MESH = plsc.VectorSubcoreMesh(core_axis_name="c", subcore_axis_name="s",
                               num_cores=NC, num_subcores=NS)

NBUF = 5


def _make_ring_agg(width, gather):
    vec = (D,) if width == D else ()
    scratch = [pltpu.VMEM((NCH, CH), jnp.int32)]
    if gather:
        scratch.insert(0, pltpu.VMEM((EPT,), jnp.int32))
        scratch.append(pltpu.VMEM((PAD,), jnp.float32))
    nrows = NBUF if gather else 1
    scratch += [pltpu.VMEM((CH,) + vec, jnp.float32) for _ in range(nrows)]
    scratch.append(pltpu.VMEM_SHARED((PAD,) + vec, jnp.float32))
    nsem = NBUF if gather else 1
    scratch += [pltpu.SemaphoreType.DMA for _ in range(nsem)]

    def body(*refs):
        if gather:
            (src_hbm, dst_hbm, table_hbm, zeros_hbm, out_hbm, sidx, didx,
             table_vm, *rest) = refs
            rows = rest[:NBUF]
            acc = rest[NBUF]
            ssem = rest[NBUF + 1:]
        else:
            (dst_hbm, zeros_hbm, out_hbm, didx, rows1, acc, ssem1) = refs
        c = lax.axis_index("c")
        s = lax.axis_index("s")
        wid = c * NS + s
        pltpu.sync_copy(zeros_hbm.at[pl.ds(s * STRIPE, STRIPE)],
                        acc.at[pl.ds(s * STRIPE, STRIPE)])
        pltpu.sync_copy(dst_hbm.at[wid], didx)

        if gather:
            pltpu.sync_copy(src_hbm.at[pl.ds(wid * EPT, EPT)], sidx)
            pltpu.sync_copy(table_hbm, table_vm)
            plsc.subcore_barrier()

            def ring(r, carry):
                for b in range(NBUF):
                    j = r * NBUF + b

                    @pl.when(r > 0)
                    def _():
                        pltpu.make_async_copy(rows[b], acc.at[didx.at[0]],
                                              ssem[b]).wait()
                    for k in range(CH // 16):
                        iv = sidx[pl.ds(j * CH + k * 16, 16)]
                        vals = plsc.load_gather(table_vm, [iv])
                        rows[b][pl.ds(k * 16, 16)] = vals
                    pltpu.async_copy(rows[b], acc.at[didx.at[j]],
                                     ssem[b], add=True)
                return carry

            lax.fori_loop(0, NCH // NBUF, ring, 0)
            for b in range(NBUF):
                pltpu.make_async_copy(rows[b], acc.at[didx.at[0]],
                                      ssem[b]).wait()
        else:
            for i in range(CH // 16):
                rows1[pl.ds(i * 16, 16)] = jnp.ones((16,), jnp.float32)
            plsc.subcore_barrier()

            def fire(j, carry):
                pltpu.async_copy(rows1, acc.at[didx.at[j]], ssem1,
                                 add=True)

                @pl.when(j >= NBUF)
                def _():
                    pltpu.make_async_copy(rows1, acc.at[didx.at[0]],
                                          ssem1).wait()
                return carry

            lax.fori_loop(0, NCH, fire, 0)
            for _ in range(NBUF):
                pltpu.make_async_copy(rows1, acc.at[didx.at[0]],
                                      ssem1).wait()
        plsc.subcore_barrier()
        pltpu.sync_copy(acc.at[pl.ds(s * STRIPE, STRIPE)],
                        out_hbm.at[c, pl.ds(s * STRIPE, STRIPE)])

    return pl.kernel(
        body,
        out_type=jax.ShapeDtypeStruct((NC, PAD) + vec, jnp.float32),
        mesh=_MESH,
        scratch_types=scratch,
        compiler_params=pltpu.CompilerParams(needs_layout_passes=False),
    )


_deg_count = _make_ring_agg(1, gather=False)
_agg_scalar = _make_ring_agg(1, gather=True)

CH2 = 80
NCH2 = EPT // CH2
NB2 = 3
NTAIL = NCH2 - (NCH2 // (2 * NB2)) * 2 * NB2


def _rows_agg_body(idx2_hbm, table_hbm, zeros_hbm, out_hbm, *rest):
    idxA = rest[:NB2]
    idxB = rest[NB2:2 * NB2]
    rows = rest[2 * NB2:3 * NB2]
    acc = rest[3 * NB2]
    r0 = 3 * NB2 + 1
    isemA = rest[r0:r0 + NB2]
    isemB = rest[r0 + NB2:r0 + 2 * NB2]
    gsem = rest[r0 + 2 * NB2:r0 + 3 * NB2]
    ssem = rest[r0 + 3 * NB2:]
    c = lax.axis_index("c")
    s = lax.axis_index("s")
    wid = c * NS + s
    for b in range(NB2):
        pltpu.async_copy(idx2_hbm.at[wid, b], idxA[b], isemA[b])
        pltpu.async_copy(idx2_hbm.at[wid, NB2 + b], idxB[b], isemB[b])
    pltpu.sync_copy(zeros_hbm.at[pl.ds(s * STRIPE, STRIPE)],
                    acc.at[pl.ds(s * STRIPE, STRIPE)])
    for b in range(NB2):
        pltpu.make_async_copy(idx2_hbm.at[wid, 0], idxA[b],
                              isemA[b]).wait()
        pltpu.async_copy(table_hbm.at[idxA[b].at[0]], rows[b], gsem[b])
    plsc.subcore_barrier()

    def ring(i, carry):
        base = i * 2 * NB2
        for b in range(NB2):
            pltpu.make_async_copy(table_hbm.at[idxA[b].at[0]], rows[b],
                                  gsem[b]).wait()
            pltpu.async_copy(rows[b], acc.at[idxA[b].at[1]], ssem[b],
                             add=True)
        for b in range(NB2):
            pltpu.make_async_copy(idx2_hbm.at[wid, 0], idxB[b],
                                  isemB[b]).wait()
            pltpu.make_async_copy(rows[b], acc.at[idxA[b].at[1]],
                                  ssem[b]).wait()
            pltpu.async_copy(table_hbm.at[idxB[b].at[0]], rows[b],
                             gsem[b])
            jf = base + 2 * NB2 + b

            @pl.when(jf < NCH2)
            def _():
                pltpu.async_copy(idx2_hbm.at[wid, jf], idxA[b], isemA[b])
        for b in range(NB2):
            pltpu.make_async_copy(table_hbm.at[idxB[b].at[0]], rows[b],
                                  gsem[b]).wait()
            pltpu.async_copy(rows[b], acc.at[idxB[b].at[1]], ssem[b],
                             add=True)
        for b in range(NB2):
            jn = base + 2 * NB2 + b

            @pl.when(jn < NCH2)
            def _():
                pltpu.make_async_copy(idx2_hbm.at[wid, 0], idxA[b],
                                      isemA[b]).wait()
                pltpu.make_async_copy(rows[b], acc.at[idxB[b].at[1]],
                                      ssem[b]).wait()
                pltpu.async_copy(table_hbm.at[idxA[b].at[0]], rows[b],
                                 gsem[b])
                jf = base + 3 * NB2 + b

                @pl.when(jf < NCH2)
                def _():
                    pltpu.async_copy(idx2_hbm.at[wid, jf], idxB[b],
                                     isemB[b])
        return carry

    lax.fori_loop(0, NCH2 // (2 * NB2), ring, 0)
    for b in range(NB2):
        pltpu.make_async_copy(table_hbm.at[idxA[b].at[0]], rows[b],
                              gsem[b]).wait()
        pltpu.async_copy(rows[b], acc.at[idxA[b].at[1]], ssem[b],
                         add=True)
    for b in range(NTAIL - NB2):
        pltpu.make_async_copy(idx2_hbm.at[wid, 0], idxB[b],
                              isemB[b]).wait()
        pltpu.make_async_copy(rows[b], acc.at[idxA[b].at[1]],
                              ssem[b]).wait()
        pltpu.async_copy(table_hbm.at[idxB[b].at[0]], rows[b], gsem[b])
    for b in range(NTAIL - NB2):
        pltpu.make_async_copy(table_hbm.at[idxB[b].at[0]], rows[b],
                              gsem[b]).wait()
        pltpu.async_copy(rows[b], acc.at[idxB[b].at[1]], ssem[b],
                         add=True)
    for b in range(NTAIL - NB2):
        pltpu.make_async_copy(rows[b], acc.at[idxB[b].at[1]],
                              ssem[b]).wait()
    for b in range(NTAIL - NB2, NB2):
        pltpu.make_async_copy(rows[b], acc.at[idxA[b].at[1]],
                              ssem[b]).wait()
    plsc.subcore_barrier()
    pltpu.sync_copy(acc.at[pl.ds(s * STRIPE, STRIPE)],
                    out_hbm.at[c, pl.ds(s * STRIPE, STRIPE)])


_agg_rows = pl.kernel(
    _rows_agg_body,
    out_type=jax.ShapeDtypeStruct((NC, PAD, D), jnp.float32),
    mesh=_MESH,
    scratch_types=(
        [pltpu.VMEM((2, CH2), jnp.int32) for _ in range(2 * NB2)]
        + [pltpu.VMEM((CH2, D), jnp.float32) for _ in range(NB2)]
        + [pltpu.VMEM_SHARED((PAD, D), jnp.float32)]
        + [pltpu.SemaphoreType.DMA for _ in range(4 * NB2)]
    ),
)


def _tc_pre(x_ref, w1_ref, cnt_ref, xw_ref, xws_ref):
    xw = jnp.dot(x_ref[...], w1_ref[...], preferred_element_type=jnp.float32)
    cnt = cnt_ref[0, 0, 0, :] + cnt_ref[1, 0, 0, :]
    dinv = lax.rsqrt(cnt + 1.0)
    xw_ref[...] = xw
    xws_ref[...] = xw * dinv[:, None]


def _tc_mid(p_ref, xw_ref, cnt_ref, b1_ref, w2_ref, wlin_ref, z_ref, zs_ref):
    agg = p_ref[0] + p_ref[1]
    cnt = cnt_ref[0, 0, 0, :] + cnt_ref[1, 0, 0, :]
    deg = cnt + 1.0
    dinv = lax.rsqrt(deg)
    h1 = agg * dinv[:, None] + xw_ref[...] * (1.0 / deg)[:, None] + b1_ref[...]
    h1 = jnp.maximum(h1, 0.0)
    w2l = jnp.dot(w2_ref[...], wlin_ref[...], preferred_element_type=jnp.float32)
    z = jnp.dot(h1, w2l, preferred_element_type=jnp.float32)
    z_ref[...] = z
    zs_ref[...] = z * dinv[:, None]


def _tc_post(az_ref, z_ref, cnt_ref, b2_ref, wlin_ref, blin_ref, out_ref):
    az = az_ref[0, 0, 0, :] + az_ref[1, 0, 0, :]
    cnt = cnt_ref[0, 0, 0, :] + cnt_ref[1, 0, 0, :]
    deg = cnt + 1.0
    dinv = lax.rsqrt(deg)
    c0 = jnp.sum(b2_ref[...] * wlin_ref[..., 0]) + blin_ref[0]
    pre = az * dinv + z_ref[..., 0] * (1.0 / deg) + c0
    out_ref[...] = jax.nn.sigmoid(pre)[:, None]


def kernel(x, edge_index, W1, b1, W2, b2, Wlin, blin):
    ei = edge_index.astype(jnp.int32)
    src_r = ei[0].reshape(NW, NCH, CH)
    dst_r = ei[1].reshape(NW, NCH, CH)
    idx2 = jnp.stack([ei[0].reshape(NW, NCH2, CH2),
                      ei[1].reshape(NW, NCH2, CH2)], axis=2)
    zeros1 = jnp.zeros((PAD,), jnp.float32)
    zerosD = jnp.zeros((PAD, D), jnp.float32)

    cnt = _deg_count(dst_r, zeros1)
    cnt_r = cnt[:, :N].reshape(NC, GRID, 1, BLK)
    cnt_spec = pl.BlockSpec((NC, 1, 1, BLK), lambda i: (0, i, 0, 0))

    xw, xws = pl.pallas_call(
        _tc_pre,
        grid=(GRID,),
        in_specs=[
            pl.BlockSpec((BLK, D), lambda i: (i, 0)),
            pl.BlockSpec((D, D), lambda i: (0, 0)),
            cnt_spec,
        ],
        out_specs=[
            pl.BlockSpec((BLK, D), lambda i: (i, 0)),
            pl.BlockSpec((BLK, D), lambda i: (i, 0)),
        ],
        out_shape=[
            jax.ShapeDtypeStruct((N, D), jnp.float32),
            jax.ShapeDtypeStruct((N, D), jnp.float32),
        ],
    )(x, W1, cnt_r)

    p = _agg_rows(idx2, xws, zerosD)

    z, zs = pl.pallas_call(
        _tc_mid,
        grid=(GRID,),
        in_specs=[
            pl.BlockSpec((NC, BLK, D), lambda i: (0, i, 0)),
            pl.BlockSpec((BLK, D), lambda i: (i, 0)),
            cnt_spec,
            pl.BlockSpec((D,), lambda i: (0,)),
            pl.BlockSpec((D, D), lambda i: (0, 0)),
            pl.BlockSpec((D, 1), lambda i: (0, 0)),
        ],
        out_specs=[
            pl.BlockSpec((BLK, 1), lambda i: (i, 0)),
            pl.BlockSpec((BLK, 1), lambda i: (i, 0)),
        ],
        out_shape=[
            jax.ShapeDtypeStruct((N, 1), jnp.float32),
            jax.ShapeDtypeStruct((N, 1), jnp.float32),
        ],
    )(p, xw, cnt_r, b1, W2, Wlin)

    zs_pad = jnp.concatenate([zs.reshape(N),
                              jnp.zeros((PAD - N,), jnp.float32)])
    az = _agg_scalar(ei[0], dst_r, zs_pad, zeros1)
    az_r = az[:, :N].reshape(NC, GRID, 1, BLK)

    out = pl.pallas_call(
        _tc_post,
        grid=(GRID,),
        in_specs=[
            cnt_spec,
            pl.BlockSpec((BLK, 1), lambda i: (i, 0)),
            cnt_spec,
            pl.BlockSpec((D,), lambda i: (0,)),
            pl.BlockSpec((D, 1), lambda i: (0, 0)),
            pl.BlockSpec((1,), lambda i: (0,)),
        ],
        out_specs=pl.BlockSpec((BLK, 1), lambda i: (i, 0)),
        out_shape=jax.ShapeDtypeStruct((N, 1), jnp.float32),
    )(az_r, z, cnt_r, b2, Wlin, blin)

    return out


# speedup vs baseline: 1.0275x; 1.0006x over previous
import jax
import jax.numpy as jnp
from jax import lax
from jax.experimental import pallas as pl
from jax.experimental.pallas import tpu as pltpu
from jax.experimental.pallas import tpu_sc as plsc

N = 10000
D = 128
E = 320000
NC, NS = 2, 16
NW = NC * NS
EPT = E // NW
CH = 80
NCH = EPT // CH
PAD = 10240
STRIPE = PAD // NS
BLK = 1000
GRID = N // BLK


_MESH = plsc.VectorSubcoreMesh(core_axis_name="c", subcore_axis_name="s",
                               num_cores=NC, num_subcores=NS)

NBUF = 5


def _make_ring_agg(width, gather):
    vec = (D,) if width == D else ()
    scratch = [pltpu.VMEM((NCH, CH), jnp.int32)]
    if gather:
        scratch.insert(0, pltpu.VMEM((EPT,), jnp.int32))
        scratch.append(pltpu.VMEM((PAD,), jnp.float32))
    nrows = NBUF if gather else 1
    scratch += [pltpu.VMEM((CH,) + vec, jnp.float32) for _ in range(nrows)]
    scratch.append(pltpu.VMEM_SHARED((PAD,) + vec, jnp.float32))
    nsem = NBUF if gather else 1
    scratch += [pltpu.SemaphoreType.DMA for _ in range(nsem)]

    def body(*refs):
        if gather:
            (src_hbm, dst_hbm, table_hbm, zeros_hbm, out_hbm, sidx, didx,
             table_vm, *rest) = refs
            rows = rest[:NBUF]
            acc = rest[NBUF]
            ssem = rest[NBUF + 1:]
        else:
            (dst_hbm, zeros_hbm, out_hbm, didx, rows1, acc, ssem1) = refs
        c = lax.axis_index("c")
        s = lax.axis_index("s")
        wid = c * NS + s
        pltpu.sync_copy(zeros_hbm.at[pl.ds(s * STRIPE, STRIPE)],
                        acc.at[pl.ds(s * STRIPE, STRIPE)])
        pltpu.sync_copy(dst_hbm.at[wid], didx)

        if gather:
            pltpu.sync_copy(src_hbm.at[pl.ds(wid * EPT, EPT)], sidx)
            pltpu.sync_copy(table_hbm, table_vm)
            plsc.subcore_barrier()

            def ring(r, carry):
                for b in range(NBUF):
                    j = r * NBUF + b

                    @pl.when(r > 0)
                    def _():
                        pltpu.make_async_copy(rows[b], acc.at[didx.at[0]],
                                              ssem[b]).wait()
                    for k in range(CH // 16):
                        iv = sidx[pl.ds(j * CH + k * 16, 16)]
                        vals = plsc.load_gather(table_vm, [iv])
                        rows[b][pl.ds(k * 16, 16)] = vals
                    pltpu.async_copy(rows[b], acc.at[didx.at[j]],
                                     ssem[b], add=True)
                return carry

            lax.fori_loop(0, NCH // NBUF, ring, 0)
            for b in range(NBUF):
                pltpu.make_async_copy(rows[b], acc.at[didx.at[0]],
                                      ssem[b]).wait()
        else:
            for i in range(CH // 16):
                rows1[pl.ds(i * 16, 16)] = jnp.ones((16,), jnp.float32)
            plsc.subcore_barrier()

            def fire(j, carry):
                pltpu.async_copy(rows1, acc.at[didx.at[j]], ssem1,
                                 add=True)

                @pl.when(j >= NBUF)
                def _():
                    pltpu.make_async_copy(rows1, acc.at[didx.at[0]],
                                          ssem1).wait()
                return carry

            lax.fori_loop(0, NCH, fire, 0)
            for _ in range(NBUF):
                pltpu.make_async_copy(rows1, acc.at[didx.at[0]],
                                      ssem1).wait()
        plsc.subcore_barrier()
        pltpu.sync_copy(acc.at[pl.ds(s * STRIPE, STRIPE)],
                        out_hbm.at[c, pl.ds(s * STRIPE, STRIPE)])

    return pl.kernel(
        body,
        out_type=jax.ShapeDtypeStruct((NC, PAD) + vec, jnp.float32),
        mesh=_MESH,
        scratch_types=scratch,
        compiler_params=pltpu.CompilerParams(needs_layout_passes=False),
    )


_deg_count = _make_ring_agg(1, gather=False)
_agg_scalar = _make_ring_agg(1, gather=True)

CH2 = 80
NCH2 = EPT // CH2
NB2 = 3
NTAIL = NCH2 - (NCH2 // (2 * NB2)) * 2 * NB2


def _rows_agg_body(idx2_hbm, table_hbm, zeros_hbm, out_hbm, *rest):
    idxA = rest[:NB2]
    idxB = rest[NB2:2 * NB2]
    rows = rest[2 * NB2:3 * NB2]
    acc = rest[3 * NB2]
    r0 = 3 * NB2 + 1
    isemA = rest[r0:r0 + NB2]
    isemB = rest[r0 + NB2:r0 + 2 * NB2]
    gsem = rest[r0 + 2 * NB2:r0 + 3 * NB2]
    ssem = rest[r0 + 3 * NB2:]
    c = lax.axis_index("c")
    s = lax.axis_index("s")
    wid = c * NS + s
    for b in range(NB2):
        pltpu.async_copy(idx2_hbm.at[wid, b], idxA[b], isemA[b])
        pltpu.async_copy(idx2_hbm.at[wid, NB2 + b], idxB[b], isemB[b])
    pltpu.sync_copy(zeros_hbm.at[pl.ds(s * STRIPE, STRIPE)],
                    acc.at[pl.ds(s * STRIPE, STRIPE)])
    for b in range(NB2):
        pltpu.make_async_copy(idx2_hbm.at[wid, 0], idxA[b],
                              isemA[b]).wait()
        pltpu.async_copy(table_hbm.at[idxA[b].at[0]], rows[b], gsem[b])
    plsc.subcore_barrier()

    def ring(i, carry):
        base = i * 2 * NB2
        for b in range(NB2):
            pltpu.make_async_copy(table_hbm.at[idxA[b].at[0]], rows[b],
                                  gsem[b]).wait()
            pltpu.async_copy(rows[b], acc.at[idxA[b].at[1]], ssem[b],
                             add=True)
        for b in range(NB2):
            pltpu.make_async_copy(idx2_hbm.at[wid, 0], idxB[b],
                                  isemB[b]).wait()
            pltpu.make_async_copy(rows[b], acc.at[idxA[b].at[1]],
                                  ssem[b]).wait()
            pltpu.async_copy(table_hbm.at[idxB[b].at[0]], rows[b],
                             gsem[b])
            jf = base + 2 * NB2 + b

            @pl.when(jf < NCH2)
            def _():
                pltpu.async_copy(idx2_hbm.at[wid, jf], idxA[b], isemA[b])
        for b in range(NB2):
            pltpu.make_async_copy(table_hbm.at[idxB[b].at[0]], rows[b],
                                  gsem[b]).wait()
            pltpu.async_copy(rows[b], acc.at[idxB[b].at[1]], ssem[b],
                             add=True)
        for b in range(NB2):
            jn = base + 2 * NB2 + b

            @pl.when(jn < NCH2)
            def _():
                pltpu.make_async_copy(idx2_hbm.at[wid, 0], idxA[b],
                                      isemA[b]).wait()
                pltpu.make_async_copy(rows[b], acc.at[idxB[b].at[1]],
                                      ssem[b]).wait()
                pltpu.async_copy(table_hbm.at[idxA[b].at[0]], rows[b],
                                 gsem[b])
                jf = base + 3 * NB2 + b

                @pl.when(jf < NCH2)
                def _():
                    pltpu.async_copy(idx2_hbm.at[wid, jf], idxB[b],
                                     isemB[b])
        return carry

    lax.fori_loop(0, NCH2 // (2 * NB2), ring, 0)
    for b in range(NB2):
        pltpu.make_async_copy(table_hbm.at[idxA[b].at[0]], rows[b],
                              gsem[b]).wait()
        pltpu.async_copy(rows[b], acc.at[idxA[b].at[1]], ssem[b],
                         add=True)
    for b in range(NTAIL - NB2):
        pltpu.make_async_copy(idx2_hbm.at[wid, 0], idxB[b],
                              isemB[b]).wait()
        pltpu.make_async_copy(rows[b], acc.at[idxA[b].at[1]],
                              ssem[b]).wait()
        pltpu.async_copy(table_hbm.at[idxB[b].at[0]], rows[b], gsem[b])
    for b in range(NTAIL - NB2):
        pltpu.make_async_copy(table_hbm.at[idxB[b].at[0]], rows[b],
                              gsem[b]).wait()
        pltpu.async_copy(rows[b], acc.at[idxB[b].at[1]], ssem[b],
                         add=True)
    for b in range(NTAIL - NB2):
        pltpu.make_async_copy(rows[b], acc.at[idxB[b].at[1]],
                              ssem[b]).wait()
    for b in range(NTAIL - NB2, NB2):
        pltpu.make_async_copy(rows[b], acc.at[idxA[b].at[1]],
                              ssem[b]).wait()
    plsc.subcore_barrier()
    pltpu.sync_copy(acc.at[pl.ds(s * STRIPE, STRIPE)],
                    out_hbm.at[c, pl.ds(s * STRIPE, STRIPE)])


_agg_rows = pl.kernel(
    _rows_agg_body,
    out_type=jax.ShapeDtypeStruct((NC, PAD, D), jnp.float32),
    mesh=_MESH,
    scratch_types=(
        [pltpu.VMEM((2, CH2), jnp.int32) for _ in range(2 * NB2)]
        + [pltpu.VMEM((CH2, D), jnp.float32) for _ in range(NB2)]
        + [pltpu.VMEM_SHARED((PAD, D), jnp.float32)]
        + [pltpu.SemaphoreType.DMA for _ in range(4 * NB2)]
    ),
)


def _tc_pre(x_ref, w1_ref, cnt_ref, xw_ref, xws_ref):
    xw = jnp.dot(x_ref[...], w1_ref[...], preferred_element_type=jnp.float32)
    cnt = cnt_ref[0, 0, 0, :] + cnt_ref[1, 0, 0, :]
    dinv = lax.rsqrt(cnt + 1.0)
    xw_ref[...] = xw
    xws_ref[...] = xw * dinv[:, None]


def _tc_mid(p_ref, xw_ref, cnt_ref, b1_ref, w2_ref, wlin_ref, z_ref, zs_ref):
    agg = p_ref[0] + p_ref[1]
    cnt = cnt_ref[0, 0, 0, :] + cnt_ref[1, 0, 0, :]
    deg = cnt + 1.0
    dinv = lax.rsqrt(deg)
    h1 = agg * dinv[:, None] + xw_ref[...] * (1.0 / deg)[:, None] + b1_ref[...]
    h1 = jnp.maximum(h1, 0.0)
    w2l = jnp.dot(w2_ref[...], wlin_ref[...], preferred_element_type=jnp.float32)
    z = jnp.dot(h1, w2l, preferred_element_type=jnp.float32)
    z_ref[...] = z
    zs_ref[...] = z * dinv[:, None]


def _tc_post(az_ref, z_ref, cnt_ref, b2_ref, wlin_ref, blin_ref, out_ref):
    az = az_ref[0, 0, 0, :] + az_ref[1, 0, 0, :]
    cnt = cnt_ref[0, 0, 0, :] + cnt_ref[1, 0, 0, :]
    deg = cnt + 1.0
    dinv = lax.rsqrt(deg)
    c0 = jnp.sum(b2_ref[...] * wlin_ref[..., 0]) + blin_ref[0]
    pre = az * dinv + z_ref[..., 0] * (1.0 / deg) + c0
    out_ref[...] = jax.nn.sigmoid(pre)[:, None]


def kernel(x, edge_index, W1, b1, W2, b2, Wlin, blin):
    ei = edge_index.astype(jnp.int32)
    src_r = ei[0].reshape(NW, NCH, CH)
    dst_r = ei[1].reshape(NW, NCH, CH)
    idx2 = jnp.stack([ei[0].reshape(NW, NCH2, CH2),
                      ei[1].reshape(NW, NCH2, CH2)], axis=2)
    zeros1 = jnp.zeros((PAD,), jnp.float32)
    zerosD = jnp.zeros((PAD, D), jnp.float32)

    cnt = _deg_count(dst_r, zeros1)
    cnt_r = cnt[:, :N].reshape(NC, GRID, 1, BLK)
    cnt_spec = pl.BlockSpec((NC, 1, 1, BLK), lambda i: (0, i, 0, 0))

    xw, xws = pl.pallas_call(
        _tc_pre,
        grid=(GRID,),
        in_specs=[
            pl.BlockSpec((BLK, D), lambda i: (i, 0)),
            pl.BlockSpec((D, D), lambda i: (0, 0)),
            cnt_spec,
        ],
        out_specs=[
            pl.BlockSpec((BLK, D), lambda i: (i, 0)),
            pl.BlockSpec((BLK, D), lambda i: (i, 0)),
        ],
        out_shape=[
            jax.ShapeDtypeStruct((N, D), jnp.float32),
            jax.ShapeDtypeStruct((N, D), jnp.float32),
        ],
    )(x, W1, cnt_r)

    p = _agg_rows(idx2, xws, zerosD)

    z, zs = pl.pallas_call(
        _tc_mid,
        grid=(GRID,),
        in_specs=[
            pl.BlockSpec((NC, BLK, D), lambda i: (0, i, 0)),
            pl.BlockSpec((BLK, D), lambda i: (i, 0)),
            cnt_spec,
            pl.BlockSpec((D,), lambda i: (0,)),
            pl.BlockSpec((D, D), lambda i: (0, 0)),
            pl.BlockSpec((D, 1), lambda i: (0, 0)),
        ],
        out_specs=[
            pl.BlockSpec((BLK, 1), lambda i: (i, 0)),
            pl.BlockSpec((BLK, 1), lambda i: (i, 0)),
        ],
        out_shape=[
            jax.ShapeDtypeStruct((N, 1), jnp.float32),
            jax.ShapeDtypeStruct((N, 1), jnp.float32),
        ],
    )(p, xw, cnt_r, b1, W2, Wlin)

    zs_pad = jnp.concatenate([zs.reshape(N),
                              jnp.zeros((PAD - N,), jnp.float32)])
    az = _agg_scalar(ei[0], dst_r, zs_pad, zeros1)
    az_r = az[:, :N].reshape(NC, GRID, 1, BLK)

    out = pl.pallas_call(
        _tc_post,
        grid=(GRID,),
        in_specs=[
            cnt_spec,
            pl.BlockSpec((BLK, 1), lambda i: (i, 0)),
            cnt_spec,
            pl.BlockSpec((D,), lambda i: (0,)),
            pl.BlockSpec((D, 1), lambda i: (0, 0)),
            pl.BlockSpec((1,), lambda i: (0,)),
        ],
        out_specs=pl.BlockSpec((BLK, 1), lambda i: (i, 0)),
        out_shape=jax.ShapeDtypeStruct((N, 1), jnp.float32),
    )(az_r, z, cnt_r, b2, Wlin, blin)

    return out
